# Initial kernel scaffold; baseline (speedup 1.0000x reference)
import jax
import jax.numpy as jnp
from jax.experimental import pallas as pl


def kernel(cls, cls_emb):
    raise NotImplementedError("write your pallas kernel here")


# speedup vs baseline: 1.4969x; 1.4969x over previous
import jax
import jax.numpy as jnp
from jax.experimental import pallas as pl
from jax.experimental.pallas import tpu as pltpu


_BATCH_TILE = 256


def _gather_kernel(cls_smem, emb_ref, o_ref):
    tb = o_ref.shape[0]
    base = pl.program_id(0) * tb
    for mi in range(tb):
        idx = cls_smem[base + mi]
        o_ref[mi, 0] = emb_ref[idx, 0]


def kernel(cls, cls_emb):
    cls_shape = cls.shape
    batch = 1
    for d in cls_shape:
        batch *= d
    n_class, cond_dim = cls_emb.shape
    out_dtype = cls_emb.dtype

    cls_i32 = jnp.clip(cls.reshape(batch).astype(jnp.int32), 0, n_class - 1)

    tb = min(_BATCH_TILE, batch)
    padded_batch = ((batch + tb - 1) // tb) * tb
    if padded_batch != batch:
        cls_i32 = jnp.pad(cls_i32, (0, padded_batch - batch))

    emb3 = cls_emb.reshape(n_class, 1, cond_dim)

    table_bytes = n_class * cond_dim * jnp.dtype(out_dtype).itemsize
    vmem_limit = min(
        table_bytes + 4 * tb * cond_dim * jnp.dtype(out_dtype).itemsize
        + 4 * 1024 * 1024,
        64 * 1024 * 1024,
    )

    out = pl.pallas_call(
        _gather_kernel,
        out_shape=jax.ShapeDtypeStruct((padded_batch, 1, cond_dim), out_dtype),
        grid_spec=pltpu.PrefetchScalarGridSpec(
            num_scalar_prefetch=1,
            grid=(padded_batch // tb,),
            in_specs=[
                pl.BlockSpec((n_class, 1, cond_dim), lambda i, c: (0, 0, 0),
                             pipeline_mode=pl.Buffered(1)),
            ],
            out_specs=pl.BlockSpec((tb, 1, cond_dim), lambda i, c: (i, 0, 0)),
        ),
        compiler_params=pltpu.CompilerParams(
            dimension_semantics=("parallel",),
            vmem_limit_bytes=int(vmem_limit)),
    )(cls_i32, emb3)

    if padded_batch != batch:
        out = out[:batch]
    return out.reshape(*cls_shape, cond_dim)
